# Initial kernel scaffold; baseline (speedup 1.0000x reference)
import jax
import jax.numpy as jnp
from jax.experimental import pallas as pl


def kernel(x, edge_index, Wl1, bl1, Wr1, br1, att1, bias1, Wl2, bl2, Wr2, br2, att2, bias2):
    raise NotImplementedError("write your pallas kernel here")


# speedup vs baseline: 17.3378x; 17.3378x over previous
import functools

import jax
import jax.numpy as jnp
from jax import lax
from jax.experimental import pallas as pl
from jax.experimental.pallas import tpu as pltpu
from jax.experimental.pallas import tpu_sc as plsc

N = 10000
E = 160000
D_IN = 256
ET = E + N
NW = 32
EPT = 5632
EP = NW * EPT
CE = 128
NCHUNK = EPT // CE
NP = 10240
RPT = NP // 16
DUMP = 10200
NPR = NP // 128
W1R = 2 * NPR + 96
W2R = 2 * NPR + 96
NPH = NP // 2
RPT2 = NPH // 16
EPT1 = EP // 16
NCHUNK1 = EPT1 // CE
EPS = 1e-16


def _mm1_body(x_ref, wl_ref, bl_ref, wr_ref, br_ref, xl_ref, xr_ref):
    xv = x_ref[...]
    xl_ref[...] = jnp.dot(xv, wl_ref[...], preferred_element_type=jnp.float32) + bl_ref[...]
    xr_ref[...] = jnp.dot(xv, wr_ref[...], preferred_element_type=jnp.float32) + br_ref[...]


def _mm1(x, Wl, bl, Wr, br):
    BM = 1000
    return pl.pallas_call(
        _mm1_body,
        grid=(N // BM,),
        in_specs=[
            pl.BlockSpec((BM, D_IN), lambda i: (i, 0)),
            pl.BlockSpec((D_IN, 128), lambda i: (0, 0)),
            pl.BlockSpec((1, 128), lambda i: (0, 0)),
            pl.BlockSpec((D_IN, 128), lambda i: (0, 0)),
            pl.BlockSpec((1, 128), lambda i: (0, 0)),
        ],
        out_specs=[
            pl.BlockSpec((BM, 128), lambda i: (i, 0)),
            pl.BlockSpec((BM, 128), lambda i: (i, 0)),
        ],
        out_shape=[jax.ShapeDtypeStruct((N, 128), jnp.float32)] * 2,
    )(x, Wl, bl.reshape(1, -1), Wr, br.reshape(1, -1))


def _mid_body(a0_ref, a1_ref, ws_ref, r_ref, bias1_ref,
              wl2_ref, bl2_ref, wr2_ref, br2_ref, yl_ref, yr_ref):
    num = jnp.concatenate([a0_ref[...], a1_ref[...]], axis=1)
    srep = jnp.dot(ws_ref[...], r_ref[...],
                   preferred_element_type=jnp.float32)
    h = num / (srep + EPS) + bias1_ref[...]
    h = jnp.where(h > 0, h, jnp.exp(jnp.minimum(h, 0.0)) - 1.0)
    yl_ref[...] = jnp.dot(h, wl2_ref[...], preferred_element_type=jnp.float32) + bl2_ref[...]
    yr_ref[...] = jnp.dot(h, wr2_ref[...], preferred_element_type=jnp.float32) + br2_ref[...]


def _mid(a0, a1, ws, rmat, bias1, Wl2, bl2, Wr2, br2):
    BM = 1000
    return pl.pallas_call(
        _mid_body,
        grid=(N // BM,),
        in_specs=[
            pl.BlockSpec((BM, 64), lambda i: (i, 0)),
            pl.BlockSpec((BM, 64), lambda i: (i, 0)),
            pl.BlockSpec((BM, 4), lambda i: (i, 0)),
            pl.BlockSpec((4, 128), lambda i: (0, 0)),
            pl.BlockSpec((1, 128), lambda i: (0, 0)),
            pl.BlockSpec((128, 1), lambda i: (0, 0)),
            pl.BlockSpec((1, 1), lambda i: (0, 0)),
            pl.BlockSpec((128, 1), lambda i: (0, 0)),
            pl.BlockSpec((1, 1), lambda i: (0, 0)),
        ],
        out_specs=[
            pl.BlockSpec((BM, 1), lambda i: (i, 0)),
            pl.BlockSpec((BM, 1), lambda i: (i, 0)),
        ],
        out_shape=[jax.ShapeDtypeStruct((N, 1), jnp.float32)] * 2,
    )(a0, a1, ws, rmat, bias1.reshape(1, -1),
      Wl2, bl2.reshape(1, 1), Wr2, br2.reshape(1, 1))


def _final_body(bb_ref, s_ref, bias2_ref, out_ref):
    nd = jnp.dot(bb_ref[...], s_ref[...],
                 preferred_element_type=jnp.float32)
    out_ref[...] = nd[:, 0:1] / (nd[:, 1:2] + EPS) + bias2_ref[...]


def _final(bb, smat, bias2):
    BM = 1000
    return pl.pallas_call(
        _final_body,
        grid=(N // BM,),
        in_specs=[
            pl.BlockSpec((BM, 4), lambda i: (i, 0)),
            pl.BlockSpec((4, 2), lambda i: (0, 0)),
            pl.BlockSpec((1, 1), lambda i: (0, 0)),
        ],
        out_specs=pl.BlockSpec((BM, 1), lambda i: (i, 0)),
        out_shape=jax.ShapeDtypeStruct((N, 1), jnp.float32),
    )(bb, smat, bias2.reshape(1, 1))


def _iota16():
    return lax.broadcasted_iota(jnp.int32, (16,), 0)


def _lane_sum(v):
    iota = _iota16()
    s = v
    for k in (8, 4, 2, 1):
        s = s + s.at[iota ^ k].get(mode="promise_in_bounds")
    return s


def _lane_bcast(v, i):
    return v.at[jnp.full((16,), i, jnp.int32)].get(mode="promise_in_bounds")


def _edge1_body(xl_hbm, xr_hbm, src_hbm, dst_hbm, att_hbm,
                out1_hbm, out2_hbm,
                srcv, dstv, gdstv, d2v, idxv, xlv, xrv, outv, attv, wsum,
                acc, wacc, sem1, sem2):
    cid = lax.axis_index("c")
    sid = lax.axis_index("s")
    iota = _iota16()

    pltpu.sync_copy(att_hbm, attv)

    def zrow(r, carry):
        for j in range(8):
            outv[r, pl.ds(16 * j, 16)] = jnp.zeros((16,), jnp.float32)
        return carry
    lax.fori_loop(0, CE, zrow, 0)

    def zw(r, carry):
        for j in range(8):
            wsum[r, pl.ds(16 * j, 16)] = jnp.zeros((16,), jnp.float32)
        return carry
    lax.fori_loop(0, W1R, zw, 0)

    for k in range(RPT2 // CE):
        pltpu.sync_copy(outv, acc.at[pl.ds(sid * RPT2 + k * CE, CE)])
    pltpu.sync_copy(outv.at[pl.ds(0, RPT2 % CE)],
                    acc.at[pl.ds(sid * RPT2 + (RPT2 // CE) * CE, RPT2 % CE)])

    @pl.when(sid == 0)
    def _():
        for k in range(W1R // CE):
            pltpu.sync_copy(outv, wacc.at[pl.ds(k * CE, CE)])
    plsc.subcore_barrier()

    att_regs = [attv[pl.ds(cid * 64 + 16 * j, 16)] for j in range(4)]

    def chunk(t, carry):
        base = sid * EPT1 + t * CE
        pltpu.sync_copy(src_hbm.at[pl.ds(base, CE)], srcv)
        pltpu.sync_copy(dst_hbm.at[pl.ds(base, CE)], dstv)
        def clampg(g, c):
            d = dstv[pl.ds(g * 16, 16)]
            gdstv[pl.ds(g * 16, 16)] = jnp.minimum(d, N - 1)
            d2v[pl.ds(g * 16, 16)] = d // 2
            return c
        lax.fori_loop(0, CE // 16, clampg, 0)
        cp1 = pltpu.async_copy(xl_hbm.at[srcv], xlv, sem1)
        cp2 = pltpu.async_copy(xr_hbm.at[gdstv], xrv, sem2)
        cp1.wait()
        cp2.wait()

        def edge(e, ecarry):
            els = []
            ts = []
            for j in range(4):
                el = xlv[e, pl.ds(cid * 64 + 16 * j, 16)]
                er = xrv[e, pl.ds(cid * 64 + 16 * j, 16)]
                u = el + er
                u = jnp.maximum(u, 0.2 * u)
                els.append(el)
                ts.append(u * att_regs[j])
            svec = jnp.zeros((16,), jnp.float32)
            for h in range(2):
                sh = _lane_sum(ts[2 * h] + ts[2 * h + 1])
                svec = jnp.where(iota == h, sh, svec)
            wvec = jnp.where(iota < 2, jnp.exp(svec), 0.0)
            d16 = plsc.load_gather(dstv, [jnp.full((16,), e, jnp.int32)])
            pf = (d16 % 2).astype(jnp.float32)
            mlo = 1.0 - pf
            for h in range(2):
                wh = _lane_bcast(wvec, h)
                for j in range(2):
                    v = els[2 * h + j] * wh
                    outv[e, pl.ds(32 * h + 16 * j, 16)] = v * mlo
                    outv[e, pl.ds(64 + 32 * h + 16 * j, 16)] = v * pf
            dr = d16 // 128
            dc = d16 % 128
            idxr = jnp.where(iota < 2, iota * NPR + dr, 2 * NPR)
            idxc = jnp.where(iota < 2, dc, iota)
            plsc.addupdate_scatter(wsum, [idxr, idxc], wvec)
            return ecarry
        lax.fori_loop(0, CE, edge, 0)
        pltpu.sync_copy(outv, acc.at[d2v], add=True)
        return carry
    lax.fori_loop(0, NCHUNK1, chunk, 0)
    for k in range(W1R // CE):
        for g in range(CE // 16):
            idxv[pl.ds(16 * g, 16)] = iota + (k * CE + 16 * g)
        pltpu.sync_copy(wsum.at[pl.ds(k * CE, CE)], wacc.at[idxv], add=True)
    plsc.subcore_barrier()

    for k in range(RPT2 // CE):
        r0 = sid * RPT2 + k * CE
        pltpu.sync_copy(acc.at[pl.ds(r0, CE)], outv)
        pltpu.sync_copy(outv, out1_hbm.at[cid, pl.ds(r0, CE)])
    r0 = sid * RPT2 + (RPT2 // CE) * CE
    pltpu.sync_copy(acc.at[pl.ds(r0, RPT2 % CE)], outv.at[pl.ds(0, RPT2 % CE)])
    pltpu.sync_copy(outv.at[pl.ds(0, RPT2 % CE)],
                    out1_hbm.at[cid, pl.ds(r0, RPT2 % CE)])
    r1 = sid * (W1R // 16)
    pltpu.sync_copy(wacc.at[pl.ds(r1, W1R // 16)], outv.at[pl.ds(0, W1R // 16)])
    pltpu.sync_copy(outv.at[pl.ds(0, W1R // 16)],
                    out2_hbm.at[cid, pl.ds(r1, W1R // 16)])


def _edge2_body(yl_hbm, yr_hbm, src_hbm, dst_hbm, att2_hbm, out_hbm,
                ylv, yrv, srcv, dstv, idxv, tbl, att2v, wacc):
    cid = lax.axis_index("c")
    sid = lax.axis_index("s")
    wid = cid * 16 + sid
    iota = _iota16()

    pltpu.sync_copy(yl_hbm, ylv)
    pltpu.sync_copy(yr_hbm, yrv)
    pltpu.sync_copy(att2_hbm, att2v)
    att2f = att2v[pl.ds(0, 16)]

    def zw(r, carry):
        for j in range(8):
            tbl[r, pl.ds(16 * j, 16)] = jnp.zeros((16,), jnp.float32)
        return carry
    lax.fori_loop(0, W2R, zw, 0)

    @pl.when(sid == 0)
    def _():
        for k in range(W2R // CE):
            pltpu.sync_copy(tbl.at[pl.ds(0, CE)], wacc.at[pl.ds(k * CE, CE)])
    plsc.subcore_barrier()

    def chunk(t, carry):
        base = wid * EPT + t * CE
        pltpu.sync_copy(src_hbm.at[pl.ds(base, CE)], srcv)
        pltpu.sync_copy(dst_hbm.at[pl.ds(base, CE)], dstv)

        def group(g, gcarry):
            s16 = srcv[pl.ds(g * 16, 16)]
            d16 = dstv[pl.ds(g * 16, 16)]
            d16c = jnp.minimum(d16, N - 1)
            ylg = plsc.load_gather(ylv, [s16])
            yrg = plsc.load_gather(yrv, [d16c])
            u = ylg + yrg
            u = jnp.maximum(u, 0.2 * u)
            w = jnp.exp(u * att2f)
            dr = d16 // 128
            dc = d16 % 128
            plsc.addupdate_scatter(tbl, [dr, dc], w * ylg)
            plsc.addupdate_scatter(tbl, [NPR + dr, dc], w)
            return gcarry
        lax.fori_loop(0, CE // 16, group, 0)
        return carry
    lax.fori_loop(0, NCHUNK, chunk, 0)

    for k in range(W2R // CE):
        for g in range(CE // 16):
            idxv[pl.ds(16 * g, 16)] = iota + (k * CE + 16 * g)
        pltpu.sync_copy(tbl.at[pl.ds(k * CE, CE)], wacc.at[idxv], add=True)
    plsc.subcore_barrier()
    r1 = sid * (W2R // 16)
    pltpu.sync_copy(wacc.at[pl.ds(r1, W2R // 16)], tbl.at[pl.ds(0, W2R // 16)])
    pltpu.sync_copy(tbl.at[pl.ds(0, W2R // 16)],
                    out_hbm.at[cid, pl.ds(r1, W2R // 16)])


@functools.cache
def _make_edge_kernels():
    mesh = plsc.VectorSubcoreMesh(core_axis_name="c", subcore_axis_name="s")
    cparams = pltpu.CompilerParams(needs_layout_passes=False)
    edge1 = functools.partial(
        pl.kernel,
        mesh=mesh,
        compiler_params=cparams,
        out_type=[
            pltpu.HBM((2, NPH, 128), jnp.float32),
            pltpu.HBM((2, W1R, 128), jnp.float32),
        ],
        scratch_types=[
            pltpu.VMEM((CE,), jnp.int32),
            pltpu.VMEM((CE,), jnp.int32),
            pltpu.VMEM((CE,), jnp.int32),
            pltpu.VMEM((CE,), jnp.int32),
            pltpu.VMEM((CE,), jnp.int32),
            pltpu.VMEM((CE, 128), jnp.float32),
            pltpu.VMEM((CE, 128), jnp.float32),
            pltpu.VMEM((CE, 128), jnp.float32),
            pltpu.VMEM((128,), jnp.float32),
            pltpu.VMEM((W1R, 128), jnp.float32),
            pltpu.VMEM_SHARED((NPH, 128), jnp.float32),
            pltpu.VMEM_SHARED((W1R, 128), jnp.float32),
            pltpu.SemaphoreType.DMA,
            pltpu.SemaphoreType.DMA,
        ],
    )(_edge1_body)
    edge2 = functools.partial(
        pl.kernel,
        mesh=mesh,
        compiler_params=cparams,
        out_type=jax.ShapeDtypeStruct((2, W2R, 128), jnp.float32),
        scratch_types=[
            pltpu.VMEM((N,), jnp.float32),
            pltpu.VMEM((N,), jnp.float32),
            pltpu.VMEM((CE,), jnp.int32),
            pltpu.VMEM((CE,), jnp.int32),
            pltpu.VMEM((CE,), jnp.int32),
            pltpu.VMEM((W2R, 128), jnp.float32),
            pltpu.VMEM((16,), jnp.float32),
            pltpu.VMEM_SHARED((W2R, 128), jnp.float32),
        ],
    )(_edge2_body)
    return edge1, edge2


def kernel(x, edge_index, Wl1, bl1, Wr1, br1, att1, bias1,
           Wl2, bl2, Wr2, br2, att2, bias2):
    edge1, edge2 = _make_edge_kernels()
    loop = jnp.arange(N, dtype=edge_index.dtype)
    pad_src = jnp.zeros((EP - ET,), dtype=edge_index.dtype)
    pad_dst = jnp.full((EP - ET,), DUMP, dtype=edge_index.dtype)
    src = jnp.concatenate([edge_index[0], loop, pad_src])
    dst = jnp.concatenate([edge_index[1], loop, pad_dst])

    cidx = jnp.arange(128, dtype=jnp.int32)
    ridx = jnp.arange(4, dtype=jnp.int32)
    rmat = (ridx[:, None] == cidx[None, :] // 32).astype(jnp.float32)
    sidx = jnp.arange(4, dtype=jnp.int32)
    smat = jnp.stack([(sidx % 2 == 0).astype(jnp.float32),
                      (sidx % 2 == 1).astype(jnp.float32)], axis=1)

    xl, xr = _mm1(x, Wl1, bl1, Wr1, br1)
    a, w1 = edge1(xl, xr, src, dst, att1.reshape(-1))
    ar = a.reshape(2, NP, 64)
    ws = (w1.reshape(2, W1R * 128)[:, :2 * NP].reshape(2, 2, NP)
          .transpose(2, 0, 1).reshape(NP, 4)[:N])
    yl, yr = _mid(ar[0, :N], ar[1, :N], ws, rmat, bias1, Wl2, bl2, Wr2, br2)
    b = edge2(yl.reshape(-1), yr.reshape(-1), src, dst,
              jnp.broadcast_to(att2.reshape(1), (16,)))
    bb = (b.reshape(2, W2R * 128)[:, :2 * NP].reshape(2, 2, NP)
          .transpose(2, 0, 1).reshape(NP, 4)[:N])
    return _final(bb, smat, bias2)


# speedup vs baseline: 18.2772x; 1.0542x over previous
import functools

import jax
import jax.numpy as jnp
from jax import lax
from jax.experimental import pallas as pl
from jax.experimental.pallas import tpu as pltpu
from jax.experimental.pallas import tpu_sc as plsc

N = 10000
E = 160000
D_IN = 256
ET = E + N
NW = 32
EPT = 5632
EP = NW * EPT
CE = 128
NCHUNK = EPT // CE
NP = 10240
DUMP = 10200
NPR = NP // 128
W1R = 2 * NPR + 96
W2R = 2 * NPR + 96
EPS16 = EP // 16
NCHUNK1 = EPS16 // CE
NP2 = 6144
RPT2 = NP2 // 16
EPS = 1e-16


def _mm1_body(x_ref, wl_ref, bl_ref, wr_ref, br_ref, xl_ref, xr_ref):
    xv = x_ref[...]
    xl_ref[...] = jnp.dot(xv, wl_ref[...], preferred_element_type=jnp.float32) + bl_ref[...]
    xr_ref[...] = jnp.dot(xv, wr_ref[...], preferred_element_type=jnp.float32) + br_ref[...]


def _mm1(x, Wl, bl, Wr, br):
    BM = 1000
    return pl.pallas_call(
        _mm1_body,
        grid=(N // BM,),
        in_specs=[
            pl.BlockSpec((BM, D_IN), lambda i: (i, 0)),
            pl.BlockSpec((D_IN, 128), lambda i: (0, 0)),
            pl.BlockSpec((1, 128), lambda i: (0, 0)),
            pl.BlockSpec((D_IN, 128), lambda i: (0, 0)),
            pl.BlockSpec((1, 128), lambda i: (0, 0)),
        ],
        out_specs=[
            pl.BlockSpec((BM, 128), lambda i: (i, 0)),
            pl.BlockSpec((BM, 128), lambda i: (i, 0)),
        ],
        out_shape=[jax.ShapeDtypeStruct((N, 128), jnp.float32)] * 2,
    )(x, Wl, bl.reshape(1, -1), Wr, br.reshape(1, -1))


def _mid_body(a0_ref, ws0_ref, r_ref, bias1_ref,
              wl2_ref, bl2_ref, wr2_ref, br2_ref, yl_ref, yr_ref):
    num = a0_ref[...]
    srep = jnp.dot(ws0_ref[...], r_ref[...],
                   preferred_element_type=jnp.float32)
    h = num / (srep + EPS) + bias1_ref[...]
    h = jnp.where(h > 0, h, jnp.exp(jnp.minimum(h, 0.0)) - 1.0)
    yl_ref[...] = jnp.dot(h, wl2_ref[...], preferred_element_type=jnp.float32) + bl2_ref[...]
    yr_ref[...] = jnp.dot(h, wr2_ref[...], preferred_element_type=jnp.float32) + br2_ref[...]


def _mid(a0, ws0, rmat, bias1, Wl2, bl2, Wr2, br2):
    BM = 1000
    return pl.pallas_call(
        _mid_body,
        grid=(N // BM,),
        in_specs=[
            pl.BlockSpec((BM, 128), lambda i: (i, 0)),
            pl.BlockSpec((BM, 4), lambda i: (i, 0)),
            pl.BlockSpec((4, 128), lambda i: (0, 0)),
            pl.BlockSpec((1, 128), lambda i: (0, 0)),
            pl.BlockSpec((128, 1), lambda i: (0, 0)),
            pl.BlockSpec((1, 1), lambda i: (0, 0)),
            pl.BlockSpec((128, 1), lambda i: (0, 0)),
            pl.BlockSpec((1, 1), lambda i: (0, 0)),
        ],
        out_specs=[
            pl.BlockSpec((BM, 1), lambda i: (i, 0)),
            pl.BlockSpec((BM, 1), lambda i: (i, 0)),
        ],
        out_shape=[jax.ShapeDtypeStruct((N, 1), jnp.float32)] * 2,
    )(a0, ws0, rmat, bias1.reshape(1, -1),
      Wl2, bl2.reshape(1, 1), Wr2, br2.reshape(1, 1))


def _final_body(bb_ref, s_ref, bias2_ref, out_ref):
    nd = jnp.dot(bb_ref[...], s_ref[...],
                 preferred_element_type=jnp.float32)
    out_ref[...] = nd[:, 0:1] / (nd[:, 1:2] + EPS) + bias2_ref[...]


def _final(bb, smat, bias2):
    BM = 1000
    return pl.pallas_call(
        _final_body,
        grid=(N // BM,),
        in_specs=[
            pl.BlockSpec((BM, 4), lambda i: (i, 0)),
            pl.BlockSpec((4, 2), lambda i: (0, 0)),
            pl.BlockSpec((1, 1), lambda i: (0, 0)),
        ],
        out_specs=pl.BlockSpec((BM, 1), lambda i: (i, 0)),
        out_shape=jax.ShapeDtypeStruct((N, 1), jnp.float32),
    )(bb, smat, bias2.reshape(1, 1))


def _iota16():
    return lax.broadcasted_iota(jnp.int32, (16,), 0)


def _lane_sum(v):
    iota = _iota16()
    s = v
    for k in (8, 4, 2, 1):
        s = s + s.at[iota ^ k].get(mode="promise_in_bounds")
    return s


def _lane_bcast(v, i):
    return v.at[jnp.full((16,), i, jnp.int32)].get(mode="promise_in_bounds")


def _edge1_body(xl_hbm, xr_hbm, src_hbm, dst_hbm, att_hbm,
                out1_hbm, out2_hbm,
                srcv, dstv, gdstv, idxv, xlv, xrv, attv, wsum,
                acc, wacc, sem1, sem2):
    cid = lax.axis_index("c")
    sid = lax.axis_index("s")
    iota = _iota16()

    pltpu.sync_copy(att_hbm, attv)

    def zrow(r, carry):
        for j in range(8):
            xlv[r, pl.ds(16 * j, 16)] = jnp.zeros((16,), jnp.float32)
        return carry
    lax.fori_loop(0, CE, zrow, 0)

    def zw(r, carry):
        for j in range(8):
            wsum[r, pl.ds(16 * j, 16)] = jnp.zeros((16,), jnp.float32)
        return carry
    lax.fori_loop(0, W1R, zw, 0)

    for k in range(RPT2 // CE):
        pltpu.sync_copy(xlv, acc.at[pl.ds(sid * RPT2 + k * CE, CE)])

    @pl.when(sid == 0)
    def _():
        for k in range(W1R // CE):
            pltpu.sync_copy(xlv, wacc.at[pl.ds(k * CE, CE)])
    plsc.subcore_barrier()

    att_regs = [attv[pl.ds(64 * cid + 16 * j, 16)] for j in range(4)]

    def chunk(t, carry):
        base = sid * EPS16 + t * CE
        pltpu.sync_copy(src_hbm.at[pl.ds(base, CE)], srcv)
        pltpu.sync_copy(dst_hbm.at[pl.ds(base, CE)], dstv)
        def clampg(g, c):
            d = dstv[pl.ds(g * 16, 16)]
            gdstv[pl.ds(g * 16, 16)] = jnp.minimum(d, N - 1)
            idxv[pl.ds(g * 16, 16)] = d // 2
            return c
        lax.fori_loop(0, CE // 16, clampg, 0)
        cp1 = pltpu.async_copy(xl_hbm.at[srcv], xlv, sem1)
        cp2 = pltpu.async_copy(xr_hbm.at[gdstv], xrv, sem2)
        cp1.wait()
        cp2.wait()

        def edge(e, ecarry):
            els = []
            ts = []
            for j in range(4):
                el = xlv[e, pl.ds(64 * cid + 16 * j, 16)]
                er = xrv[e, pl.ds(64 * cid + 16 * j, 16)]
                u = el + er
                u = jnp.maximum(u, 0.2 * u)
                els.append(el)
                ts.append(u * att_regs[j])
            svec = jnp.zeros((16,), jnp.float32)
            for hh in range(2):
                sh = _lane_sum(ts[2 * hh] + ts[2 * hh + 1])
                svec = jnp.where(iota == hh, sh, svec)
            wvec = jnp.where(iota < 2, jnp.exp(svec), 0.0)
            d16 = plsc.load_gather(dstv, [jnp.full((16,), e, jnp.int32)])
            evenm = (d16 % 2) == 0
            for hh in range(2):
                wh = _lane_bcast(wvec, hh)
                for j2 in range(2):
                    payload = els[2 * hh + j2] * wh
                    xrv[e, pl.ds(32 * hh + 16 * j2, 16)] = (
                        jnp.where(evenm, payload, 0.0))
                    xrv[e, pl.ds(64 + 32 * hh + 16 * j2, 16)] = (
                        jnp.where(evenm, 0.0, payload))
            dr = d16 // 128
            dc = d16 % 128
            idxr = jnp.where(iota < 2, iota * NPR + dr, 2 * NPR)
            idxc = jnp.where(iota < 2, dc, iota)
            plsc.addupdate_scatter(wsum, [idxr, idxc], wvec)
            return ecarry
        lax.fori_loop(0, CE, edge, 0)
        pltpu.sync_copy(xrv, acc.at[idxv], add=True)
        return carry
    lax.fori_loop(0, NCHUNK1, chunk, 0)
    for k in range(W1R // CE):
        for g in range(CE // 16):
            idxv[pl.ds(16 * g, 16)] = iota + (k * CE + 16 * g)
        pltpu.sync_copy(wsum.at[pl.ds(k * CE, CE)], wacc.at[idxv], add=True)
    plsc.subcore_barrier()

    for k in range(RPT2 // CE):
        pltpu.sync_copy(acc.at[pl.ds(sid * RPT2 + k * CE, CE)], xlv)
        pltpu.sync_copy(xlv, out1_hbm.at[pl.ds(cid * NP2 + sid * RPT2 + k * CE, CE)])

    r1 = sid * (W1R // 16)
    pltpu.sync_copy(wacc.at[pl.ds(r1, W1R // 16)], xlv.at[pl.ds(0, W1R // 16)])
    pltpu.sync_copy(xlv.at[pl.ds(0, W1R // 16)],
                    out2_hbm.at[cid, pl.ds(r1, W1R // 16)])


def _edge2_body(yl_hbm, yr_hbm, src_hbm, dst_hbm, att2_hbm, out_hbm,
                ylv, yrv, srcv, dstv, idxv, tbl, att2v, wacc):
    cid = lax.axis_index("c")
    sid = lax.axis_index("s")
    wid = cid * 16 + sid
    iota = _iota16()

    pltpu.sync_copy(yl_hbm, ylv)
    pltpu.sync_copy(yr_hbm, yrv)
    pltpu.sync_copy(att2_hbm, att2v)
    att2f = att2v[pl.ds(0, 16)]

    def zw(r, carry):
        for j in range(8):
            tbl[r, pl.ds(16 * j, 16)] = jnp.zeros((16,), jnp.float32)
        return carry
    lax.fori_loop(0, W2R, zw, 0)

    @pl.when(sid == 0)
    def _():
        for k in range(W2R // CE):
            pltpu.sync_copy(tbl.at[pl.ds(0, CE)], wacc.at[pl.ds(k * CE, CE)])
    plsc.subcore_barrier()

    def chunk(t, carry):
        base = wid * EPT + t * CE
        pltpu.sync_copy(src_hbm.at[pl.ds(base, CE)], srcv)
        pltpu.sync_copy(dst_hbm.at[pl.ds(base, CE)], dstv)

        def group(g, gcarry):
            s16 = srcv[pl.ds(g * 16, 16)]
            d16 = dstv[pl.ds(g * 16, 16)]
            d16c = jnp.minimum(d16, N - 1)
            ylg = plsc.load_gather(ylv, [s16])
            yrg = plsc.load_gather(yrv, [d16c])
            u = ylg + yrg
            u = jnp.maximum(u, 0.2 * u)
            w = jnp.exp(u * att2f)
            dr = d16 // 128
            dc = d16 % 128
            plsc.addupdate_scatter(tbl, [dr, dc], w * ylg)
            plsc.addupdate_scatter(tbl, [NPR + dr, dc], w)
            return gcarry
        lax.fori_loop(0, CE // 16, group, 0)
        return carry
    lax.fori_loop(0, NCHUNK, chunk, 0)

    for k in range(W2R // CE):
        for g in range(CE // 16):
            idxv[pl.ds(16 * g, 16)] = iota + (k * CE + 16 * g)
        pltpu.sync_copy(tbl.at[pl.ds(k * CE, CE)], wacc.at[idxv], add=True)
    plsc.subcore_barrier()
    r1 = sid * (W2R // 16)
    pltpu.sync_copy(wacc.at[pl.ds(r1, W2R // 16)], tbl.at[pl.ds(0, W2R // 16)])
    pltpu.sync_copy(tbl.at[pl.ds(0, W2R // 16)],
                    out_hbm.at[cid, pl.ds(r1, W2R // 16)])


@functools.cache
def _make_edge_kernels():
    mesh = plsc.VectorSubcoreMesh(core_axis_name="c", subcore_axis_name="s")
    cparams = pltpu.CompilerParams(needs_layout_passes=False)
    edge1 = functools.partial(
        pl.kernel,
        mesh=mesh,
        compiler_params=cparams,
        out_type=[
            pltpu.HBM((2 * NP2, 128), jnp.float32),
            pltpu.HBM((2, W1R, 128), jnp.float32),
        ],
        scratch_types=[
            pltpu.VMEM((CE,), jnp.int32),
            pltpu.VMEM((CE,), jnp.int32),
            pltpu.VMEM((CE,), jnp.int32),
            pltpu.VMEM((CE,), jnp.int32),
            pltpu.VMEM((CE, 128), jnp.float32),
            pltpu.VMEM((CE, 128), jnp.float32),
            pltpu.VMEM((128,), jnp.float32),
            pltpu.VMEM((W1R, 128), jnp.float32),
            pltpu.VMEM_SHARED((NP2, 128), jnp.float32),
            pltpu.VMEM_SHARED((W1R, 128), jnp.float32),
            pltpu.SemaphoreType.DMA,
            pltpu.SemaphoreType.DMA,
        ],
    )(_edge1_body)
    edge2 = functools.partial(
        pl.kernel,
        mesh=mesh,
        compiler_params=cparams,
        out_type=jax.ShapeDtypeStruct((2, W2R, 128), jnp.float32),
        scratch_types=[
            pltpu.VMEM((N,), jnp.float32),
            pltpu.VMEM((N,), jnp.float32),
            pltpu.VMEM((CE,), jnp.int32),
            pltpu.VMEM((CE,), jnp.int32),
            pltpu.VMEM((CE,), jnp.int32),
            pltpu.VMEM((W2R, 128), jnp.float32),
            pltpu.VMEM((16,), jnp.float32),
            pltpu.VMEM_SHARED((W2R, 128), jnp.float32),
        ],
    )(_edge2_body)
    return edge1, edge2


def kernel(x, edge_index, Wl1, bl1, Wr1, br1, att1, bias1,
           Wl2, bl2, Wr2, br2, att2, bias2):
    edge1, edge2 = _make_edge_kernels()
    loop = jnp.arange(N, dtype=edge_index.dtype)
    pad_src = jnp.zeros((EP - ET,), dtype=edge_index.dtype)
    pad_dst = jnp.full((EP - ET,), DUMP, dtype=edge_index.dtype)
    src = jnp.concatenate([edge_index[0], loop, pad_src])
    dst = jnp.concatenate([edge_index[1], loop, pad_dst])

    cidx = jnp.arange(128, dtype=jnp.int32)
    ridx = jnp.arange(4, dtype=jnp.int32)
    rmat = (ridx[:, None] == cidx[None, :] // 32).astype(jnp.float32)
    sidx = jnp.arange(4, dtype=jnp.int32)
    smat = jnp.stack([(sidx % 2 == 0).astype(jnp.float32),
                      (sidx % 2 == 1).astype(jnp.float32)], axis=1)

    xl, xr = _mm1(x, Wl1, bl1, Wr1, br1)
    a, w1 = edge1(xl, xr, src, dst, att1.reshape(-1))
    ws = (w1.reshape(2, W1R * 128)[:, :2 * NP].reshape(2, 2, NP)[..., :N]
          .reshape(4, N).T)
    num = (a.reshape(2, NP2, 2, 2, 32).transpose(1, 2, 0, 3, 4)
           .reshape(2 * NP2, 128)[:N])
    yl, yr = _mid(num, ws, rmat, bias1, Wl2, bl2, Wr2, br2)
    b = edge2(yl.reshape(-1), yr.reshape(-1), src, dst,
              jnp.broadcast_to(att2.reshape(1), (16,)))
    bb = (b.reshape(2, W2R * 128)[:, :2 * NP].reshape(2, 2, NP)
          .transpose(2, 0, 1).reshape(NP, 4)[:N])
    return _final(bb, smat, bias2)
